# initial kernel scaffold (unmeasured)
import jax
import jax.numpy as jnp
from jax import lax
from jax.experimental import pallas as pl
from jax.experimental.pallas import tpu as pltpu

B, SQ, H, D = 8, 8, 16, 128
SKV_LOCAL = 1024
N_SUB = 4
SKV_SUB = SKV_LOCAL // N_SUB
SCALE = D ** -0.5
COMM_DTYPE = jnp.bfloat16


def kernel(Q, K, V):
    def body(q_ref, k_hbm, v_hbm, o_ref,
             k_vmem, v_vmem, l_ref,
             o_send, o_recv, l_send, l_recv,
             load_sems, send_o_sems, recv_o_sems, send_l_sems, recv_l_sems):
        my_x = lax.axis_index("x")
        my_y = lax.axis_index("y")
        my_z = lax.axis_index("z")

        start = (my_y * 2 + my_z) * SKV_SUB
        k_load = pltpu.make_async_copy(
            k_hbm.at[:, pl.ds(start, SKV_SUB)], k_vmem, load_sems.at[0])
        v_load = pltpu.make_async_copy(
            v_hbm.at[:, pl.ds(start, SKV_SUB)], v_vmem, load_sems.at[1])
        k_load.start()
        v_load.start()

        barrier = pltpu.get_barrier_semaphore()
        neighbors = (
            (my_x, my_y, 1 - my_z),
            (my_x, 1 - my_y, my_z),
            (1 - my_x, my_y, my_z),
        )
        for nbr in neighbors:
            pl.semaphore_signal(barrier, inc=1, device_id=nbr,
                                device_id_type=pl.DeviceIdType.MESH)
        pl.semaphore_wait(barrier, 3)

        k_load.wait()
        v_load.wait()

        for b in range(B):
            qb = (q_ref[b] * SCALE).astype(jnp.bfloat16)
            kb = k_vmem[b].astype(jnp.bfloat16)
            vb = v_vmem[b].astype(jnp.bfloat16)
            s = lax.dot_general(
                qb, kb, (((2,), (2,)), ((1,), (1,))),
                preferred_element_type=jnp.float32)
            p = jnp.exp(s)
            l_ref[b] = jnp.sum(p, axis=-1)
            o = lax.dot_general(
                p.astype(jnp.bfloat16), vb, (((2,), (0,)), ((0,), (1,))),
                preferred_element_type=jnp.float32)
            o_ref[b] = jnp.transpose(o, (1, 0, 2))

        for r, peer in enumerate(neighbors):
            o_send[...] = o_ref[...].astype(COMM_DTYPE)
            l_send[...] = l_ref[...]
            o_rdma = pltpu.make_async_remote_copy(
                src_ref=o_send, dst_ref=o_recv.at[r],
                send_sem=send_o_sems.at[r], recv_sem=recv_o_sems.at[r],
                device_id=peer, device_id_type=pl.DeviceIdType.MESH)
            l_rdma = pltpu.make_async_remote_copy(
                src_ref=l_send, dst_ref=l_recv.at[r],
                send_sem=send_l_sems.at[r], recv_sem=recv_l_sems.at[r],
                device_id=peer, device_id_type=pl.DeviceIdType.MESH)
            o_rdma.start()
            l_rdma.start()
            o_rdma.wait()
            l_rdma.wait()
            o_ref[...] = o_ref[...] + o_recv[r].astype(jnp.float32)
            l_ref[...] = l_ref[...] + l_recv[r]

        linv = 1.0 / l_ref[...]
        o_ref[...] = o_ref[...] * jnp.transpose(linv, (0, 2, 1))[..., None]

    return pl.pallas_call(
        body,
        out_shape=jax.ShapeDtypeStruct((B, SQ, H, D), jnp.float32),
        in_specs=[
            pl.BlockSpec(memory_space=pltpu.VMEM),
            pl.BlockSpec(memory_space=pltpu.ANY),
            pl.BlockSpec(memory_space=pltpu.ANY),
        ],
        out_specs=pl.BlockSpec(memory_space=pltpu.VMEM),
        scratch_shapes=[
            pltpu.VMEM((B, SKV_SUB, H, D), jnp.float32),
            pltpu.VMEM((B, SKV_SUB, H, D), jnp.float32),
            pltpu.VMEM((B, H, SQ), jnp.float32),
            pltpu.VMEM((B, SQ, H, D), COMM_DTYPE),
            pltpu.VMEM((3, B, SQ, H, D), COMM_DTYPE),
            pltpu.VMEM((B, H, SQ), jnp.float32),
            pltpu.VMEM((3, B, H, SQ), jnp.float32),
            pltpu.SemaphoreType.DMA((2,)),
            pltpu.SemaphoreType.DMA((3,)),
            pltpu.SemaphoreType.DMA((3,)),
            pltpu.SemaphoreType.DMA((3,)),
            pltpu.SemaphoreType.DMA((3,)),
        ],
        compiler_params=pltpu.CompilerParams(collective_id=0),
    )(Q, K, V)


# baseline (device time: 72874 ns/iter reference)
import jax
import jax.numpy as jnp
from jax import lax
from jax.experimental import pallas as pl
from jax.experimental.pallas import tpu as pltpu

B, SQ, H, D = 8, 8, 16, 128
SKV_LOCAL = 1024
N_SUB = 4
SKV_SUB = SKV_LOCAL // N_SUB
SCALE = D ** -0.5
COMM_DTYPE = jnp.bfloat16


def kernel(Q, K, V):
    def body(q_ref, k_hbm, v_hbm, o_ref,
             k_vmem, v_vmem, l_ref,
             o_send, o_recv, l_send, l_recv,
             load_sems, send_o_sems, recv_o_sems, send_l_sems, recv_l_sems):
        my_x = lax.axis_index("x")
        my_y = lax.axis_index("y")
        my_z = lax.axis_index("z")

        start = (my_y * 2 + my_z) * SKV_SUB
        k_load = pltpu.make_async_copy(
            k_hbm.at[:, pl.ds(start, SKV_SUB)], k_vmem, load_sems.at[0])
        v_load = pltpu.make_async_copy(
            v_hbm.at[:, pl.ds(start, SKV_SUB)], v_vmem, load_sems.at[1])
        k_load.start()
        v_load.start()

        barrier = pltpu.get_barrier_semaphore()
        neighbors = (
            (my_x, my_y, 1 - my_z),
            (my_x, 1 - my_y, my_z),
            (1 - my_x, my_y, my_z),
        )
        for nbr in neighbors:
            pl.semaphore_signal(barrier, inc=1, device_id=nbr,
                                device_id_type=pl.DeviceIdType.MESH)
        pl.semaphore_wait(barrier, 3)

        k_load.wait()
        v_load.wait()

        for b in range(B):
            qb = (q_ref[b] * SCALE).astype(jnp.bfloat16)
            kb = k_vmem[b].astype(jnp.bfloat16)
            vb = v_vmem[b].astype(jnp.bfloat16)
            s = lax.dot_general(
                qb, kb, (((2,), (2,)), ((1,), (1,))),
                preferred_element_type=jnp.float32)
            p = jnp.exp(s)
            l_ref[b] = jnp.sum(p, axis=-1)
            o = lax.dot_general(
                p.astype(jnp.bfloat16), vb, (((2,), (0,)), ((0,), (1,))),
                preferred_element_type=jnp.float32)
            o_ref[b] = jnp.transpose(o, (1, 0, 2))

        for r, peer in enumerate(neighbors):
            o_send[...] = o_ref[...].astype(COMM_DTYPE)
            l_send[...] = l_ref[...]
            o_rdma = pltpu.make_async_remote_copy(
                src_ref=o_send, dst_ref=o_recv.at[r],
                send_sem=send_o_sems.at[r], recv_sem=recv_o_sems.at[r],
                device_id=peer, device_id_type=pl.DeviceIdType.MESH)
            l_rdma = pltpu.make_async_remote_copy(
                src_ref=l_send, dst_ref=l_recv.at[r],
                send_sem=send_l_sems.at[r], recv_sem=recv_l_sems.at[r],
                device_id=peer, device_id_type=pl.DeviceIdType.MESH)
            o_rdma.start()
            l_rdma.start()
            o_rdma.wait()
            l_rdma.wait()
            o_ref[...] = o_ref[...] + o_recv[r].astype(jnp.float32)
            l_ref[...] = l_ref[...] + l_recv[r]

        linv = 1.0 / l_ref[...]
        o_ref[...] = o_ref[...] * jnp.transpose(linv, (0, 2, 1))[..., None]

    return pl.pallas_call(
        body,
        out_shape=jax.ShapeDtypeStruct((B, SQ, H, D), jnp.float32),
        in_specs=[
            pl.BlockSpec(memory_space=pltpu.VMEM),
            pl.BlockSpec(memory_space=pl.ANY),
            pl.BlockSpec(memory_space=pl.ANY),
        ],
        out_specs=pl.BlockSpec(memory_space=pltpu.VMEM),
        scratch_shapes=[
            pltpu.VMEM((B, SKV_SUB, H, D), jnp.float32),
            pltpu.VMEM((B, SKV_SUB, H, D), jnp.float32),
            pltpu.VMEM((B, H, SQ), jnp.float32),
            pltpu.VMEM((B, SQ, H, D), COMM_DTYPE),
            pltpu.VMEM((3, B, SQ, H, D), COMM_DTYPE),
            pltpu.VMEM((B, H, SQ), jnp.float32),
            pltpu.VMEM((3, B, H, SQ), jnp.float32),
            pltpu.SemaphoreType.DMA((2,)),
            pltpu.SemaphoreType.DMA((3,)),
            pltpu.SemaphoreType.DMA((3,)),
            pltpu.SemaphoreType.DMA((3,)),
            pltpu.SemaphoreType.DMA((3,)),
        ],
        compiler_params=pltpu.CompilerParams(
            collective_id=0, vmem_limit_bytes=100 * 1024 * 1024),
    )(Q, K, V)


# device time: 27754 ns/iter; 2.6257x vs baseline; 2.6257x over previous
import jax
import jax.numpy as jnp
from jax import lax
from jax.experimental import pallas as pl
from jax.experimental.pallas import tpu as pltpu

B, SQ, H, D = 8, 8, 16, 128
SKV = 1024
BG = 2
SCALE = D ** -0.5
COMM_DTYPE = jnp.bfloat16


def kernel(Q, K, V):
    def body(q_ref, k_hbm, v_hbm, o_ref,
             k_vmem, v_vmem, o_part, l_part,
             po_send, po_recv, pl_send, pl_recv,
             qf_send, qf_recv,
             load_sems, p1_sems, p2_send_sems, p2_recv_sems):
        my_x = lax.axis_index("x")
        my_y = lax.axis_index("y")
        my_z = lax.axis_index("z")
        g = 2 * my_y + my_z
        b0 = BG * g

        loads = []
        for h in range(H):
            kc = pltpu.make_async_copy(
                k_hbm.at[pl.ds(b0, BG), :, h], k_vmem.at[:, h],
                load_sems.at[0, h])
            vc = pltpu.make_async_copy(
                v_hbm.at[pl.ds(b0, BG), :, h], v_vmem.at[:, h],
                load_sems.at[1, h])
            kc.start()
            vc.start()
            loads += [kc, vc]

        neighbors = (
            (1 - my_x, my_y, my_z),
            (my_x, 1 - my_y, my_z),
            (my_x, my_y, 1 - my_z),
            (my_x, 1 - my_y, 1 - my_z),
        )
        barrier = pltpu.get_barrier_semaphore()
        for nbr in neighbors:
            pl.semaphore_signal(barrier, inc=1, device_id=nbr,
                                device_id_type=pl.DeviceIdType.MESH)
        pl.semaphore_wait(barrier, 4)

        for c in loads:
            c.wait()

        q2 = jnp.transpose(q_ref[pl.ds(b0, BG)], (0, 2, 1, 3))
        for b in range(BG):
            qb = (q2[b] * SCALE).astype(jnp.bfloat16)
            kb = k_vmem[b].astype(jnp.bfloat16)
            vb = v_vmem[b].astype(jnp.bfloat16)
            s = lax.dot_general(
                qb, kb, (((2,), (2,)), ((0,), (0,))),
                preferred_element_type=jnp.float32)
            p = jnp.exp(s)
            l_part[b] = jnp.sum(p, axis=-1)
            o_part[b] = lax.dot_general(
                p.astype(jnp.bfloat16), vb, (((2,), (1,)), ((0,), (0,))),
                preferred_element_type=jnp.float32)

        po_send[...] = o_part[...].astype(COMM_DTYPE)
        pl_send[...] = l_part[...]
        o_rdma = pltpu.make_async_remote_copy(
            src_ref=po_send, dst_ref=po_recv,
            send_sem=p1_sems.at[0], recv_sem=p1_sems.at[1],
            device_id=neighbors[0], device_id_type=pl.DeviceIdType.MESH)
        l_rdma = pltpu.make_async_remote_copy(
            src_ref=pl_send, dst_ref=pl_recv,
            send_sem=p1_sems.at[2], recv_sem=p1_sems.at[3],
            device_id=neighbors[0], device_id_type=pl.DeviceIdType.MESH)
        o_rdma.start()
        l_rdma.start()
        o_rdma.wait()
        l_rdma.wait()

        o_sum = o_part[...] + po_recv[...].astype(jnp.float32)
        l_sum = l_part[...] + pl_recv[...]
        fin = o_sum / l_sum[..., None]
        t = jnp.transpose(fin, (0, 2, 1, 3))
        o_ref[pl.ds(b0, BG)] = t
        qf_send[...] = t.astype(COMM_DTYPE)

        peer_g = (2 * (1 - my_y) + my_z,
                  2 * my_y + (1 - my_z),
                  2 * (1 - my_y) + (1 - my_z))
        rdmas = []
        for i in range(3):
            r = pltpu.make_async_remote_copy(
                src_ref=qf_send, dst_ref=qf_recv.at[i],
                send_sem=p2_send_sems.at[i], recv_sem=p2_recv_sems.at[i],
                device_id=neighbors[1 + i],
                device_id_type=pl.DeviceIdType.MESH)
            r.start()
            rdmas.append(r)
        for i, r in enumerate(rdmas):
            r.wait_recv()
            o_ref[pl.ds(BG * peer_g[i], BG)] = qf_recv[i].astype(jnp.float32)
        for r in rdmas:
            r.wait_send()

    return pl.pallas_call(
        body,
        out_shape=jax.ShapeDtypeStruct((B, SQ, H, D), jnp.float32),
        in_specs=[
            pl.BlockSpec(memory_space=pltpu.VMEM),
            pl.BlockSpec(memory_space=pl.ANY),
            pl.BlockSpec(memory_space=pl.ANY),
        ],
        out_specs=pl.BlockSpec(memory_space=pltpu.VMEM),
        scratch_shapes=[
            pltpu.VMEM((BG, H, SKV, D), jnp.float32),
            pltpu.VMEM((BG, H, SKV, D), jnp.float32),
            pltpu.VMEM((BG, H, SQ, D), jnp.float32),
            pltpu.VMEM((BG, H, SQ), jnp.float32),
            pltpu.VMEM((BG, H, SQ, D), COMM_DTYPE),
            pltpu.VMEM((BG, H, SQ, D), COMM_DTYPE),
            pltpu.VMEM((BG, H, SQ), jnp.float32),
            pltpu.VMEM((BG, H, SQ), jnp.float32),
            pltpu.VMEM((BG, SQ, H, D), COMM_DTYPE),
            pltpu.VMEM((3, BG, SQ, H, D), COMM_DTYPE),
            pltpu.SemaphoreType.DMA((2, H)),
            pltpu.SemaphoreType.DMA((4,)),
            pltpu.SemaphoreType.DMA((3,)),
            pltpu.SemaphoreType.DMA((3,)),
        ],
        compiler_params=pltpu.CompilerParams(
            collective_id=0, vmem_limit_bytes=100 * 1024 * 1024),
    )(Q, K, V)


# device time: 24253 ns/iter; 3.0047x vs baseline; 1.1444x over previous
import jax
import jax.numpy as jnp
from jax import lax
from jax.experimental import pallas as pl
from jax.experimental.pallas import tpu as pltpu

B, SQ, H, D = 8, 8, 16, 128
SKV = 1024
BG = 2
HH = H // 2
UNITS = tuple((b, hh) for b in range(BG) for hh in range(2))
SCALE = D ** -0.5
COMM_DTYPE = jnp.bfloat16


def kernel(Q, K, V):
    def body(q_ref, k_hbm, v_hbm, o_ref,
             k_vmem, v_vmem, o_part, l_part,
             po_send, po_recv, pl_send, pl_recv,
             qf_send, qf_recv,
             load_sems, p1o_sems, p1l_sems, p2_send_sems, p2_recv_sems):
        my_x = lax.axis_index("x")
        my_y = lax.axis_index("y")
        my_z = lax.axis_index("z")
        g = 2 * my_y + my_z
        b0 = BG * g

        loads = {}
        for b in range(BG):
            for h in range(H):
                kc = pltpu.make_async_copy(
                    k_hbm.at[b0 + b, :, h], k_vmem.at[b, h],
                    load_sems.at[0, b, h])
                vc = pltpu.make_async_copy(
                    v_hbm.at[b0 + b, :, h], v_vmem.at[b, h],
                    load_sems.at[1, b, h])
                kc.start()
                vc.start()
                loads[(b, h)] = (kc, vc)

        neighbors = (
            (1 - my_x, my_y, my_z),
            (my_x, 1 - my_y, my_z),
            (my_x, my_y, 1 - my_z),
            (my_x, 1 - my_y, 1 - my_z),
        )
        barrier = pltpu.get_barrier_semaphore()
        for nbr in neighbors:
            pl.semaphore_signal(barrier, inc=1, device_id=nbr,
                                device_id_type=pl.DeviceIdType.MESH)
        pl.semaphore_wait(barrier, 4)

        q2 = jnp.transpose(q_ref[pl.ds(b0, BG)], (0, 2, 1, 3))

        p1_rdmas = []
        for b, hh in UNITS:
            hs = slice(hh * HH, (hh + 1) * HH)
            for h in range(hh * HH, (hh + 1) * HH):
                kc, vc = loads[(b, h)]
                kc.wait()
                vc.wait()
            qb = (q2[b, hs] * SCALE).astype(jnp.bfloat16)
            kb = k_vmem[b, hs].astype(jnp.bfloat16)
            vb = v_vmem[b, hs].astype(jnp.bfloat16)
            s = lax.dot_general(
                qb, kb, (((2,), (2,)), ((0,), (0,))),
                preferred_element_type=jnp.float32)
            p = jnp.exp(s)
            l = jnp.sum(p, axis=-1)
            o = lax.dot_general(
                p.astype(jnp.bfloat16), vb, (((2,), (1,)), ((0,), (0,))),
                preferred_element_type=jnp.float32)
            o_part[b, hs] = o
            l_part[b, hs] = l
            po_send[b, hs] = o.astype(COMM_DTYPE)
            pl_send[b, hs] = l

            u = 2 * b + hh
            o_rdma = pltpu.make_async_remote_copy(
                src_ref=po_send.at[b, pl.ds(hh * HH, HH)],
                dst_ref=po_recv.at[b, pl.ds(hh * HH, HH)],
                send_sem=p1o_sems.at[u, 0], recv_sem=p1o_sems.at[u, 1],
                device_id=neighbors[0], device_id_type=pl.DeviceIdType.MESH)
            l_rdma = pltpu.make_async_remote_copy(
                src_ref=pl_send.at[b, pl.ds(hh * HH, HH)],
                dst_ref=pl_recv.at[b, pl.ds(hh * HH, HH)],
                send_sem=p1l_sems.at[u, 0], recv_sem=p1l_sems.at[u, 1],
                device_id=neighbors[0], device_id_type=pl.DeviceIdType.MESH)
            o_rdma.start()
            l_rdma.start()
            p1_rdmas.append((o_rdma, l_rdma))

        p2_rdmas = []
        for b, hh in UNITS:
            hs = slice(hh * HH, (hh + 1) * HH)
            u = 2 * b + hh
            o_rdma, l_rdma = p1_rdmas[u]
            o_rdma.wait_recv()
            l_rdma.wait_recv()
            o_sum = o_part[b, hs] + po_recv[b, hs].astype(jnp.float32)
            l_sum = l_part[b, hs] + pl_recv[b, hs]
            fin = o_sum / l_sum[..., None]
            t = jnp.transpose(fin, (1, 0, 2))
            o_ref[b0 + b, :, hs] = t
            qf_send[b, :, hs] = t.astype(COMM_DTYPE)
            for i in range(3):
                r = pltpu.make_async_remote_copy(
                    src_ref=qf_send.at[b, :, pl.ds(hh * HH, HH)],
                    dst_ref=qf_recv.at[i, b, :, pl.ds(hh * HH, HH)],
                    send_sem=p2_send_sems.at[i, u],
                    recv_sem=p2_recv_sems.at[i, u],
                    device_id=neighbors[1 + i],
                    device_id_type=pl.DeviceIdType.MESH)
                r.start()
                p2_rdmas.append((i, b, hh, r))

        peer_g = (2 * (1 - my_y) + my_z,
                  2 * my_y + (1 - my_z),
                  2 * (1 - my_y) + (1 - my_z))
        for i, b, hh, r in p2_rdmas:
            hs = slice(hh * HH, (hh + 1) * HH)
            r.wait_recv()
            o_ref[pl.ds(BG * peer_g[i] + b, 1), :, hs] = (
                qf_recv[i, b:b + 1, :, hs].astype(jnp.float32))

        for o_rdma, l_rdma in p1_rdmas:
            o_rdma.wait_send()
            l_rdma.wait_send()
        for _, _, _, r in p2_rdmas:
            r.wait_send()

    return pl.pallas_call(
        body,
        out_shape=jax.ShapeDtypeStruct((B, SQ, H, D), jnp.float32),
        in_specs=[
            pl.BlockSpec(memory_space=pltpu.VMEM),
            pl.BlockSpec(memory_space=pl.ANY),
            pl.BlockSpec(memory_space=pl.ANY),
        ],
        out_specs=pl.BlockSpec(memory_space=pltpu.VMEM),
        scratch_shapes=[
            pltpu.VMEM((BG, H, SKV, D), jnp.float32),
            pltpu.VMEM((BG, H, SKV, D), jnp.float32),
            pltpu.VMEM((BG, H, SQ, D), jnp.float32),
            pltpu.VMEM((BG, H, SQ), jnp.float32),
            pltpu.VMEM((BG, H, SQ, D), COMM_DTYPE),
            pltpu.VMEM((BG, H, SQ, D), COMM_DTYPE),
            pltpu.VMEM((BG, H, SQ), jnp.float32),
            pltpu.VMEM((BG, H, SQ), jnp.float32),
            pltpu.VMEM((BG, SQ, H, D), COMM_DTYPE),
            pltpu.VMEM((3, BG, SQ, H, D), COMM_DTYPE),
            pltpu.SemaphoreType.DMA((2, BG, H)),
            pltpu.SemaphoreType.DMA((4, 2)),
            pltpu.SemaphoreType.DMA((4, 2)),
            pltpu.SemaphoreType.DMA((3, 4)),
            pltpu.SemaphoreType.DMA((3, 4)),
        ],
        compiler_params=pltpu.CompilerParams(
            collective_id=0, vmem_limit_bytes=100 * 1024 * 1024),
    )(Q, K, V)
